# baseline (speedup 1.0000x reference)
import functools

import jax
import jax.numpy as jnp
from jax import lax
from jax.experimental import pallas as pl
from jax.experimental.pallas import tpu as pltpu
from jax.experimental.pallas import tpu_sc as plsc

_D = 128
_CHUNK = 400
_R = 1


@functools.cache
def _build(B):
    info = plsc.get_sparse_core_info()
    nc, ns = info.num_cores, info.num_subcores
    nw = nc * ns
    b_per_w = B // nw
    n_chunks = b_per_w // _CHUNK
    n_groups = n_chunks // _R
    assert n_groups % 2 == 0 and n_groups >= 4

    mesh = plsc.VectorSubcoreMesh(core_axis_name="c", subcore_axis_name="s")

    @functools.partial(
        pl.kernel,
        out_type=jax.ShapeDtypeStruct((B, _D), jnp.float32),
        mesh=mesh,
        scratch_types=[
            pltpu.VMEM((b_per_w,), jnp.int32),
            pltpu.VMEM((2, _R, _CHUNK, _D), jnp.float32),
        ]
        + [pltpu.SemaphoreType.DMA] * (4 * _R),
    )
    def gather_kernel(pos_hbm, table_hbm, out_hbm, idx_v, rows_v, *sems):
        gsem = (sems[0:_R], sems[_R : 2 * _R])
        ssem = (sems[2 * _R : 3 * _R], sems[3 * _R : 4 * _R])
        wid = lax.axis_index("s") * nc + lax.axis_index("c")
        base = wid * b_per_w
        pltpu.sync_copy(pos_hbm.at[pl.ds(wid * b_per_w, b_per_w)], idx_v)

        def fire_gather(g, a, b):
            pltpu.async_copy(
                table_hbm.at[idx_v.at[pl.ds((g * _R + b) * _CHUNK, _CHUNK)]],
                rows_v.at[a, b],
                gsem[a][b],
            )

        def wait_gather(a, b):
            pltpu.make_async_copy(
                table_hbm.at[idx_v.at[pl.ds(b * _CHUNK, _CHUNK)]], rows_v.at[a, b], gsem[a][b]
            ).wait()

        def fire_store(g, a, b):
            pltpu.async_copy(
                rows_v.at[a, b],
                out_hbm.at[pl.ds(base + (g * _R + b) * _CHUNK, _CHUNK)],
                ssem[a][b],
            )

        def wait_store(g, a, b):
            pltpu.make_async_copy(
                rows_v.at[a, b],
                out_hbm.at[pl.ds(base + (g * _R + b) * _CHUNK, _CHUNK)],
                ssem[a][b],
            ).wait()

        for b in range(_R):
            fire_gather(0, 0, b)
        for b in range(_R):
            wait_gather(0, b)
        for b in range(_R):
            fire_store(0, 0, b)
        for b in range(_R):
            fire_gather(1, 1, b)

        def pair(t, carry):
            g_odd = 1 + 2 * t
            for b in range(_R):
                wait_gather(1, b)
            for b in range(_R):
                fire_store(g_odd, 1, b)
            for b in range(_R):
                wait_store(g_odd - 1, 0, b)
            for b in range(_R):
                fire_gather(g_odd + 1, 0, b)
            g_even = 2 + 2 * t
            for b in range(_R):
                wait_gather(0, b)
            for b in range(_R):
                fire_store(g_even, 0, b)
            for b in range(_R):
                wait_store(g_even - 1, 1, b)
            for b in range(_R):
                fire_gather(g_even + 1, 1, b)
            return carry

        lax.fori_loop(0, (n_groups - 2) // 2, pair, 0)

        g_last = n_groups - 1
        for b in range(_R):
            wait_gather(1, b)
        for b in range(_R):
            fire_store(g_last, 1, b)
        for b in range(_R):
            wait_store(g_last - 1, 0, b)
        for b in range(_R):
            wait_store(g_last, 1, b)

    return gather_kernel


def kernel(position, table):
    b0, b1 = position.shape
    B = b0 * b1
    pos2d = position.T.reshape(B).astype(jnp.int32)
    out = _build(B)(pos2d, table)
    return out.reshape(b1, b0, _D).transpose(1, 0, 2)


# speedup vs baseline: 1.0055x; 1.0055x over previous
import functools

import jax
import jax.numpy as jnp
from jax import lax
from jax.experimental import pallas as pl
from jax.experimental.pallas import tpu as pltpu
from jax.experimental.pallas import tpu_sc as plsc

_D = 128
_CHUNK = 200
_NB = 4


@functools.cache
def _build(B):
    info = plsc.get_sparse_core_info()
    nc, ns = info.num_cores, info.num_subcores
    nw = nc * ns
    b_per_w = B // nw
    n_chunks = b_per_w // _CHUNK
    assert n_chunks * _CHUNK == b_per_w and (n_chunks - _NB) % _NB == 0

    mesh = plsc.VectorSubcoreMesh(core_axis_name="c", subcore_axis_name="s")

    @functools.partial(
        pl.kernel,
        out_type=jax.ShapeDtypeStruct((B, _D), jnp.float32),
        mesh=mesh,
        scratch_types=[
            pltpu.VMEM((b_per_w,), jnp.int32),
            pltpu.VMEM((_NB, _CHUNK, _D), jnp.float32),
        ]
        + [pltpu.SemaphoreType.DMA] * (2 * _NB),
    )
    def gather_kernel(pos_hbm, table_hbm, out_hbm, idx_v, rows_v, *sems):
        gsem, ssem = sems[:_NB], sems[_NB:]
        wid = lax.axis_index("s") * nc + lax.axis_index("c")
        base = wid * b_per_w
        pltpu.sync_copy(pos_hbm.at[pl.ds(wid * b_per_w, b_per_w)], idx_v)

        def fire_gather(g, b):
            pltpu.async_copy(
                table_hbm.at[idx_v.at[pl.ds(g * _CHUNK, _CHUNK)]],
                rows_v.at[b],
                gsem[b],
            )

        def wait_gather(b):
            pltpu.make_async_copy(
                table_hbm.at[idx_v.at[pl.ds(b * _CHUNK, _CHUNK)]],
                rows_v.at[b],
                gsem[b],
            ).wait()

        def fire_store(g, b):
            pltpu.async_copy(
                rows_v.at[b],
                out_hbm.at[pl.ds(base + g * _CHUNK, _CHUNK)],
                ssem[b],
            )

        def wait_store(g, b):
            pltpu.make_async_copy(
                rows_v.at[b],
                out_hbm.at[pl.ds(base + g * _CHUNK, _CHUNK)],
                ssem[b],
            ).wait()

        fire_gather(0, 0)
        fire_gather(1, 1)
        wait_gather(0)
        fire_store(0, 0)
        fire_gather(2, 2)
        wait_gather(1)
        fire_store(1, 1)
        fire_gather(3, 3)

        def quad(t, carry):
            for i in range(_NB):
                g = 2 + _NB * t + i
                b = (2 + i) % _NB
                wait_gather(b)
                fire_store(g, b)
                wait_store(g - 2, (b + 2) % _NB)
                fire_gather(g + 2, (b + 2) % _NB)
            return carry

        lax.fori_loop(0, (n_chunks - _NB) // _NB, quad, 0)

        for g in (n_chunks - 2, n_chunks - 1):
            b = g % _NB
            wait_gather(b)
            fire_store(g, b)
            wait_store(g - 2, (b + 2) % _NB)
        wait_store(n_chunks - 2, (n_chunks - 2) % _NB)
        wait_store(n_chunks - 1, (n_chunks - 1) % _NB)

    return gather_kernel


def kernel(position, table):
    b0, b1 = position.shape
    B = b0 * b1
    pos_flat = position.T.reshape(B).astype(jnp.int32)
    out = _build(B)(pos_flat, table)
    return out.reshape(b1, b0, _D).transpose(1, 0, 2)
